# baseline (speedup 1.0000x reference)
import jax
import jax.numpy as jnp
from jax.experimental import pallas as pl
from jax.experimental.pallas import tpu as pltpu

_B = 4
_PER_STEP = 2
_NKP = 2048
_R = 128
_C = 128
_NEG = float("-inf")


def _fps_body(x_ref, y_ref, z_ref, f_ref, kp_ref, kf_ref):
    n = _R * _C
    P = _PER_STEP

    xs = [x_ref[p] for p in range(P)]
    ys = [y_ref[p] for p in range(P)]
    zs = [z_ref[p] for p in range(P)]
    fs = [f_ref[p] for p in range(P)]

    row_i = jax.lax.broadcasted_iota(jnp.int32, (_R, _C), 0)
    col_i = jax.lax.broadcasted_iota(jnp.int32, (_R, _C), 1)
    lin = row_i * _C + col_i
    lane3 = col_i[0:1, :]

    def out_row(lx, ly, lz):
        v = jnp.where(lane3 == 0, lx, jnp.where(lane3 == 1, ly, lz))
        return v[:, 0:3]

    l0 = []
    for p in range(P):
        lx = xs[p][0:1, 0:1]
        ly = ys[p][0:1, 0:1]
        lz = zs[p][0:1, 0:1]
        kp_ref[p, 0:1, :] = out_row(lx, ly, lz)
        kf_ref[p, 0:1, :] = fs[p][0:1, 0:1]
        l0.append((lx, ly, lz))

    def step(i, carry):
        dists, lxs, lys, lzs = carry
        new_dists = []
        sel = []
        for p in range(P):
            dx = xs[p] - lxs[p]
            dy = ys[p] - lys[p]
            dz = zs[p] - lzs[p]
            d = (dx * dx + dz * dz) + dy * dy
            new_dists.append(jnp.minimum(dists[p], d))
        for p in range(P):
            dist = new_dists[p]
            m = jnp.max(dist, axis=(0, 1), keepdims=True)
            k = jnp.where(dist == m, lin, jnp.int32(n))
            j = jnp.min(k, axis=(0, 1), keepdims=True)
            hit = k == j
            lx = jnp.max(jnp.where(hit, xs[p], _NEG), axis=(0, 1), keepdims=True)
            ly = jnp.max(jnp.where(hit, ys[p], _NEG), axis=(0, 1), keepdims=True)
            lz = jnp.max(jnp.where(hit, zs[p], _NEG), axis=(0, 1), keepdims=True)
            lf = jnp.max(jnp.where(hit, fs[p], _NEG), axis=(0, 1), keepdims=True)
            kp_ref[p, pl.ds(i, 1), :] = out_row(lx, ly, lz)
            kf_ref[p, pl.ds(i, 1), :] = lf
            sel.append((lx, ly, lz))
        return (
            tuple(new_dists),
            tuple(s[0] for s in sel),
            tuple(s[1] for s in sel),
            tuple(s[2] for s in sel),
        )

    dist0 = jnp.full((_R, _C), 1e10, dtype=jnp.float32)
    jax.lax.fori_loop(
        1,
        _NKP,
        step,
        (
            tuple(dist0 for _ in range(P)),
            tuple(l[0] for l in l0),
            tuple(l[1] for l in l0),
            tuple(l[2] for l in l0),
        ),
    )


def kernel(points, batch_size):
    del batch_size
    n_total = points.shape[0]
    n = n_total // _B
    pts = points.reshape(_B, n, points.shape[1])
    x = pts[:, :, 1].reshape(_B, _R, _C)
    y = pts[:, :, 2].reshape(_B, _R, _C)
    z = pts[:, :, 3].reshape(_B, _R, _C)
    f = pts[:, :, 4].reshape(_B, _R, _C)

    grid = (_B // _PER_STEP,)
    kp, kf = pl.pallas_call(
        _fps_body,
        grid=grid,
        in_specs=[
            pl.BlockSpec((_PER_STEP, _R, _C), lambda b: (b, 0, 0)),
            pl.BlockSpec((_PER_STEP, _R, _C), lambda b: (b, 0, 0)),
            pl.BlockSpec((_PER_STEP, _R, _C), lambda b: (b, 0, 0)),
            pl.BlockSpec((_PER_STEP, _R, _C), lambda b: (b, 0, 0)),
        ],
        out_specs=[
            pl.BlockSpec((_PER_STEP, _NKP, 3), lambda b: (b, 0, 0)),
            pl.BlockSpec((_PER_STEP, _NKP, 1), lambda b: (b, 0, 0)),
        ],
        out_shape=[
            jax.ShapeDtypeStruct((_B, _NKP, 3), jnp.float32),
            jax.ShapeDtypeStruct((_B, _NKP, 1), jnp.float32),
        ],
        compiler_params=pltpu.CompilerParams(
            dimension_semantics=("parallel",),
        ),
    )(x, y, z, f)
    return kp, kf


# speedup vs baseline: 3.3763x; 3.3763x over previous
import jax
import jax.numpy as jnp
from jax.experimental import pallas as pl
from jax.experimental.pallas import tpu as pltpu

_B = 4
_PER_STEP = 4
_NKP = 2048
_R = 128
_C = 128
_NEG = float("-inf")


def _fps_body(x_ref, y_ref, z_ref, f_ref, kp_ref, kf_ref):
    n = _R * _C
    P = _PER_STEP

    xs = [x_ref[p] for p in range(P)]
    ys = [y_ref[p] for p in range(P)]
    zs = [z_ref[p] for p in range(P)]
    fs = [f_ref[p] for p in range(P)]

    row_i = jax.lax.broadcasted_iota(jnp.int32, (_R, _C), 0)
    col_i = jax.lax.broadcasted_iota(jnp.int32, (_R, _C), 1)
    lin = row_i * _C + col_i
    lane3 = col_i[0:1, :]

    def out_row(lx, ly, lz):
        v = jnp.where(lane3 == 0, lx, jnp.where(lane3 == 1, ly, lz))
        return v[:, 0:3]

    l0 = []
    for p in range(P):
        lx = xs[p][0:1, 0:1]
        ly = ys[p][0:1, 0:1]
        lz = zs[p][0:1, 0:1]
        kp_ref[p, 0:1, :] = out_row(lx, ly, lz)
        kf_ref[p, 0:1, :] = fs[p][0:1, 0:1]
        l0.append((lx, ly, lz))

    def step(i, carry):
        dists, lxs, lys, lzs = carry
        new_dists = []
        sel = []
        for p in range(P):
            dx = xs[p] - lxs[p]
            dy = ys[p] - lys[p]
            dz = zs[p] - lzs[p]
            d = (dx * dx + dz * dz) + dy * dy
            new_dists.append(jnp.minimum(dists[p], d))
        for p in range(P):
            dist = new_dists[p]
            rm = jnp.max(dist, axis=0, keepdims=True)
            rowidx = jnp.min(
                jnp.where(dist == rm, row_i, jnp.int32(_R)),
                axis=0, keepdims=True,
            )
            rowhit = row_i == rowidx
            xc = jnp.max(jnp.where(rowhit, xs[p], _NEG), axis=0, keepdims=True)
            yc = jnp.max(jnp.where(rowhit, ys[p], _NEG), axis=0, keepdims=True)
            zc = jnp.max(jnp.where(rowhit, zs[p], _NEG), axis=0, keepdims=True)
            fc = jnp.max(jnp.where(rowhit, fs[p], _NEG), axis=0, keepdims=True)
            m = jnp.max(rm, axis=1, keepdims=True)
            cand = jnp.where(
                rm == m, rowidx * jnp.int32(_C) + lane3, jnp.int32(n)
            )
            j = jnp.min(cand, axis=1, keepdims=True)
            hit = cand == j
            lx = jnp.max(jnp.where(hit, xc, _NEG), axis=1, keepdims=True)
            ly = jnp.max(jnp.where(hit, yc, _NEG), axis=1, keepdims=True)
            lz = jnp.max(jnp.where(hit, zc, _NEG), axis=1, keepdims=True)
            lf = jnp.max(jnp.where(hit, fc, _NEG), axis=1, keepdims=True)
            kp_ref[p, pl.ds(i, 1), :] = out_row(lx, ly, lz)
            kf_ref[p, pl.ds(i, 1), :] = lf
            sel.append((lx, ly, lz))
        return (
            tuple(new_dists),
            tuple(s[0] for s in sel),
            tuple(s[1] for s in sel),
            tuple(s[2] for s in sel),
        )

    dist0 = jnp.full((_R, _C), 1e10, dtype=jnp.float32)
    jax.lax.fori_loop(
        1,
        _NKP,
        step,
        (
            tuple(dist0 for _ in range(P)),
            tuple(l[0] for l in l0),
            tuple(l[1] for l in l0),
            tuple(l[2] for l in l0),
        ),
    )


def kernel(points, batch_size):
    del batch_size
    n_total = points.shape[0]
    n = n_total // _B
    pts = points.reshape(_B, n, points.shape[1])
    x = pts[:, :, 1].reshape(_B, _R, _C)
    y = pts[:, :, 2].reshape(_B, _R, _C)
    z = pts[:, :, 3].reshape(_B, _R, _C)
    f = pts[:, :, 4].reshape(_B, _R, _C)

    grid = (_B // _PER_STEP,)
    kp, kf = pl.pallas_call(
        _fps_body,
        grid=grid,
        in_specs=[
            pl.BlockSpec((_PER_STEP, _R, _C), lambda b: (b, 0, 0)),
            pl.BlockSpec((_PER_STEP, _R, _C), lambda b: (b, 0, 0)),
            pl.BlockSpec((_PER_STEP, _R, _C), lambda b: (b, 0, 0)),
            pl.BlockSpec((_PER_STEP, _R, _C), lambda b: (b, 0, 0)),
        ],
        out_specs=[
            pl.BlockSpec((_PER_STEP, _NKP, 3), lambda b: (b, 0, 0)),
            pl.BlockSpec((_PER_STEP, _NKP, 1), lambda b: (b, 0, 0)),
        ],
        out_shape=[
            jax.ShapeDtypeStruct((_B, _NKP, 3), jnp.float32),
            jax.ShapeDtypeStruct((_B, _NKP, 1), jnp.float32),
        ],
        compiler_params=pltpu.CompilerParams(
            dimension_semantics=("parallel",),
        ),
    )(x, y, z, f)
    return kp, kf


# speedup vs baseline: 4.5336x; 1.3428x over previous
import jax
import jax.numpy as jnp
from jax.experimental import pallas as pl
from jax.experimental.pallas import tpu as pltpu

_B = 4
_PER_STEP = 4
_NKP = 2048
_R = 128
_C = 128
_NEG = float("-inf")


def _fps_body(x_ref, y_ref, z_ref, f_ref, kp_ref, kf_ref):
    n = _R * _C
    P = _PER_STEP

    xs = [x_ref[p] for p in range(P)]
    ys = [y_ref[p] for p in range(P)]
    zs = [z_ref[p] for p in range(P)]
    fs = [f_ref[p] for p in range(P)]

    row_i = jax.lax.broadcasted_iota(jnp.int32, (_R, _C), 0)
    col_i = jax.lax.broadcasted_iota(jnp.int32, (_R, _C), 1)
    lin = row_i * _C + col_i
    lane3 = col_i[0:1, :]

    def out_row(lx, ly, lz):
        v = jnp.where(lane3 == 0, lx, jnp.where(lane3 == 1, ly, lz))
        return v[:, 0:3]

    l0 = []
    for p in range(P):
        lx = xs[p][0:1, 0:1]
        ly = ys[p][0:1, 0:1]
        lz = zs[p][0:1, 0:1]
        kp_ref[p, 0:1, :] = out_row(lx, ly, lz)
        kf_ref[p, 0:1, :] = fs[p][0:1, 0:1]
        l0.append((lx, ly, lz))

    def step(i, carry):
        dists, lxs, lys, lzs = carry
        new_dists = []
        sel = []
        for p in range(P):
            dx = xs[p] - lxs[p]
            dy = ys[p] - lys[p]
            dz = zs[p] - lzs[p]
            d = (dx * dx + dz * dz) + dy * dy
            new_dists.append(jnp.minimum(dists[p], d))
        for p in range(P):
            dist = new_dists[p]
            rm = jnp.max(dist, axis=0, keepdims=True)
            rowidx = jnp.argmax(dist, axis=0, keepdims=True)
            rowhit = row_i == rowidx
            xc = jnp.max(jnp.where(rowhit, xs[p], _NEG), axis=0, keepdims=True)
            yc = jnp.max(jnp.where(rowhit, ys[p], _NEG), axis=0, keepdims=True)
            zc = jnp.max(jnp.where(rowhit, zs[p], _NEG), axis=0, keepdims=True)
            fc = jnp.max(jnp.where(rowhit, fs[p], _NEG), axis=0, keepdims=True)
            jl = jnp.argmax(rm, axis=1, keepdims=True)
            hit = lane3 == jl
            lx = jnp.max(jnp.where(hit, xc, _NEG), axis=1, keepdims=True)
            ly = jnp.max(jnp.where(hit, yc, _NEG), axis=1, keepdims=True)
            lz = jnp.max(jnp.where(hit, zc, _NEG), axis=1, keepdims=True)
            lf = jnp.max(jnp.where(hit, fc, _NEG), axis=1, keepdims=True)
            kp_ref[p, pl.ds(i, 1), :] = out_row(lx, ly, lz)
            kf_ref[p, pl.ds(i, 1), :] = lf
            sel.append((lx, ly, lz))
        return (
            tuple(new_dists),
            tuple(s[0] for s in sel),
            tuple(s[1] for s in sel),
            tuple(s[2] for s in sel),
        )

    dist0 = jnp.full((_R, _C), 1e10, dtype=jnp.float32)
    jax.lax.fori_loop(
        1,
        _NKP,
        step,
        (
            tuple(dist0 for _ in range(P)),
            tuple(l[0] for l in l0),
            tuple(l[1] for l in l0),
            tuple(l[2] for l in l0),
        ),
    )


def kernel(points, batch_size):
    del batch_size
    n_total = points.shape[0]
    n = n_total // _B
    pts = points.reshape(_B, n, points.shape[1])
    x = pts[:, :, 1].reshape(_B, _C, _R).swapaxes(1, 2)
    y = pts[:, :, 2].reshape(_B, _C, _R).swapaxes(1, 2)
    z = pts[:, :, 3].reshape(_B, _C, _R).swapaxes(1, 2)
    f = pts[:, :, 4].reshape(_B, _C, _R).swapaxes(1, 2)

    grid = (_B // _PER_STEP,)
    kp, kf = pl.pallas_call(
        _fps_body,
        grid=grid,
        in_specs=[
            pl.BlockSpec((_PER_STEP, _R, _C), lambda b: (b, 0, 0)),
            pl.BlockSpec((_PER_STEP, _R, _C), lambda b: (b, 0, 0)),
            pl.BlockSpec((_PER_STEP, _R, _C), lambda b: (b, 0, 0)),
            pl.BlockSpec((_PER_STEP, _R, _C), lambda b: (b, 0, 0)),
        ],
        out_specs=[
            pl.BlockSpec((_PER_STEP, _NKP, 3), lambda b: (b, 0, 0)),
            pl.BlockSpec((_PER_STEP, _NKP, 1), lambda b: (b, 0, 0)),
        ],
        out_shape=[
            jax.ShapeDtypeStruct((_B, _NKP, 3), jnp.float32),
            jax.ShapeDtypeStruct((_B, _NKP, 1), jnp.float32),
        ],
        compiler_params=pltpu.CompilerParams(
            dimension_semantics=("parallel",),
        ),
    )(x, y, z, f)
    return kp, kf


# speedup vs baseline: 4.7919x; 1.0570x over previous
import jax
import jax.numpy as jnp
from jax.experimental import pallas as pl
from jax.experimental.pallas import tpu as pltpu

_B = 4
_PER_STEP = 4
_NKP = 2048
_R = 128
_C = 128
_NEG = float("-inf")


def _fps_body(x_ref, y_ref, z_ref, f_ref, kp_ref, kf_ref, dist_ref):
    n = _R * _C
    P = _PER_STEP

    xs = [x_ref[p] for p in range(P)]
    ys = [y_ref[p] for p in range(P)]
    zs = [z_ref[p] for p in range(P)]
    fs = [f_ref[p] for p in range(P)]

    row_i = jax.lax.broadcasted_iota(jnp.int32, (_R, _C), 0)
    col_i = jax.lax.broadcasted_iota(jnp.int32, (_R, _C), 1)
    lin = row_i * _C + col_i
    lane3 = col_i[0:1, :]

    def out_row(lx, ly, lz):
        v = jnp.where(lane3 == 0, lx, jnp.where(lane3 == 1, ly, lz))
        return v[:, 0:3]

    l0 = []
    for p in range(P):
        lx = xs[p][0:1, 0:1]
        ly = ys[p][0:1, 0:1]
        lz = zs[p][0:1, 0:1]
        kp_ref[p, 0:1, :] = out_row(lx, ly, lz)
        kf_ref[p, 0:1, :] = fs[p][0:1, 0:1]
        l0.append((lx, ly, lz))

    def step(i, carry):
        lxs, lys, lzs = carry
        new_dists = []
        sel = []
        for p in range(P):
            dx = xs[p] - lxs[p]
            dy = ys[p] - lys[p]
            dz = zs[p] - lzs[p]
            d = (dx * dx + dz * dz) + dy * dy
            nd = jnp.minimum(dist_ref[p], d)
            dist_ref[p] = nd
            new_dists.append(nd)
        for p in range(P):
            dist = new_dists[p]
            rm = jnp.max(dist, axis=0, keepdims=True)
            rowidx = jnp.argmax(dist, axis=0, keepdims=True)
            rowhit = row_i == rowidx
            xc = jnp.max(jnp.where(rowhit, xs[p], _NEG), axis=0, keepdims=True)
            yc = jnp.max(jnp.where(rowhit, ys[p], _NEG), axis=0, keepdims=True)
            zc = jnp.max(jnp.where(rowhit, zs[p], _NEG), axis=0, keepdims=True)
            fc = jnp.max(jnp.where(rowhit, fs[p], _NEG), axis=0, keepdims=True)
            jl = jnp.argmax(rm, axis=1, keepdims=True)
            hit = lane3 == jl
            lx = jnp.max(jnp.where(hit, xc, _NEG), axis=1, keepdims=True)
            ly = jnp.max(jnp.where(hit, yc, _NEG), axis=1, keepdims=True)
            lz = jnp.max(jnp.where(hit, zc, _NEG), axis=1, keepdims=True)
            lf = jnp.max(jnp.where(hit, fc, _NEG), axis=1, keepdims=True)
            kp_ref[p, pl.ds(i, 1), :] = out_row(lx, ly, lz)
            kf_ref[p, pl.ds(i, 1), :] = lf
            sel.append((lx, ly, lz))
        return (
            tuple(s[0] for s in sel),
            tuple(s[1] for s in sel),
            tuple(s[2] for s in sel),
        )

    for p in range(P):
        dist_ref[p] = jnp.full((_R, _C), 1e10, dtype=jnp.float32)
    jax.lax.fori_loop(
        1,
        _NKP,
        step,
        (
            tuple(l[0] for l in l0),
            tuple(l[1] for l in l0),
            tuple(l[2] for l in l0),
        ),
    )


def kernel(points, batch_size):
    del batch_size
    n_total = points.shape[0]
    n = n_total // _B
    pts = points.reshape(_B, n, points.shape[1])
    x = pts[:, :, 1].reshape(_B, _C, _R).swapaxes(1, 2)
    y = pts[:, :, 2].reshape(_B, _C, _R).swapaxes(1, 2)
    z = pts[:, :, 3].reshape(_B, _C, _R).swapaxes(1, 2)
    f = pts[:, :, 4].reshape(_B, _C, _R).swapaxes(1, 2)

    grid = (_B // _PER_STEP,)
    kp, kf = pl.pallas_call(
        _fps_body,
        grid=grid,
        in_specs=[
            pl.BlockSpec((_PER_STEP, _R, _C), lambda b: (b, 0, 0)),
            pl.BlockSpec((_PER_STEP, _R, _C), lambda b: (b, 0, 0)),
            pl.BlockSpec((_PER_STEP, _R, _C), lambda b: (b, 0, 0)),
            pl.BlockSpec((_PER_STEP, _R, _C), lambda b: (b, 0, 0)),
        ],
        out_specs=[
            pl.BlockSpec((_PER_STEP, _NKP, 3), lambda b: (b, 0, 0)),
            pl.BlockSpec((_PER_STEP, _NKP, 1), lambda b: (b, 0, 0)),
        ],
        out_shape=[
            jax.ShapeDtypeStruct((_B, _NKP, 3), jnp.float32),
            jax.ShapeDtypeStruct((_B, _NKP, 1), jnp.float32),
        ],
        scratch_shapes=[pltpu.VMEM((_PER_STEP, _R, _C), jnp.float32)],
        compiler_params=pltpu.CompilerParams(
            dimension_semantics=("parallel",),
        ),
    )(x, y, z, f)
    return kp, kf
